# Initial kernel scaffold; baseline (speedup 1.0000x reference)
import jax
import jax.numpy as jnp
from jax.experimental import pallas as pl


def kernel(species, aevs, W0_s0, b0_s0, W1_s0, b1_s0, W2_s0, b2_s0, W0_s1, b0_s1, W1_s1, b1_s1, W2_s1, b2_s1, W0_s2, b0_s2, W1_s2, b1_s2, W2_s2, b2_s2, W0_s3, b0_s3, W1_s3, b1_s3, W2_s3, b2_s3):
    raise NotImplementedError("write your pallas kernel here")


# speedup vs baseline: 1.1583x; 1.1583x over previous
import jax
import jax.numpy as jnp
from jax.experimental import pallas as pl

N_SPECIES = 4
BLK = 512


def _celu(x):
    return jnp.maximum(x, 0.0) + jnp.minimum(
        0.0, 0.1 * (jnp.exp(jnp.minimum(x, 0.0) / 0.1) - 1.0))


def _mlp_block_kernel(spec_ref, x_ref, *refs):
    out_ref = refs[-1]
    wrefs = refs[:-1]
    x = x_ref[...]
    spec = spec_ref[0, 0, :]
    spec_b = jax.lax.broadcast_in_dim(spec, (8, spec.shape[0]), (1,))
    sids = jax.lax.broadcasted_iota(jnp.int32, (8, spec.shape[0]), 0)
    onehot = jnp.transpose(jnp.where(spec_b == sids, 1.0, 0.0))
    acc = jnp.zeros_like(out_ref)
    for s in range(N_SPECIES):
        W0, b0, W1, b1, W2, b2 = (r[...] for r in wrefs[6 * s:6 * s + 6])
        h = _celu(jax.lax.dot_general(x, W0, (((1,), (1,)), ((), ())),
                                      preferred_element_type=jnp.float32) + b0)
        h = _celu(jax.lax.dot_general(h, W1, (((1,), (1,)), ((), ())),
                                      preferred_element_type=jnp.float32) + b1)
        h = _celu(jax.lax.dot_general(h, W2, (((1,), (1,)), ((), ())),
                                      preferred_element_type=jnp.float32) + b2)
        acc = acc + h * onehot[:, s:s + 1]
    out_ref[...] = acc


def kernel(species, aevs, W0_s0, b0_s0, W1_s0, b1_s0, W2_s0, b2_s0,
           W0_s1, b0_s1, W1_s1, b1_s1, W2_s1, b2_s1,
           W0_s2, b0_s2, W1_s2, b1_s2, W2_s2, b2_s2,
           W0_s3, b0_s3, W1_s3, b1_s3, W2_s3, b2_s3):
    b, a = species.shape
    n = b * a
    aev_dim = aevs.shape[-1]
    n_feats = W2_s0.shape[0]
    nblk = n // BLK

    spec3 = species.reshape(nblk, 1, BLK)
    flat = aevs.reshape(n, aev_dim)

    weights = (W0_s0, b0_s0, W1_s0, b1_s0, W2_s0, b2_s0,
               W0_s1, b0_s1, W1_s1, b1_s1, W2_s1, b2_s1,
               W0_s2, b0_s2, W1_s2, b1_s2, W2_s2, b2_s2,
               W0_s3, b0_s3, W1_s3, b1_s3, W2_s3, b2_s3)

    def w_spec(w):
        return pl.BlockSpec(w.shape, lambda i: (0,) * w.ndim)

    out = pl.pallas_call(
        _mlp_block_kernel,
        grid=(nblk,),
        in_specs=[
            pl.BlockSpec((1, 1, BLK), lambda i: (i, 0, 0)),
            pl.BlockSpec((BLK, aev_dim), lambda i: (i, 0)),
        ] + [w_spec(w) for w in weights],
        out_specs=pl.BlockSpec((BLK, n_feats), lambda i: (i, 0)),
        out_shape=jax.ShapeDtypeStruct((n, n_feats), jnp.float32),
    )(spec3, flat, *weights)

    return species, out.reshape(b, a, n_feats)
